# initial kernel scaffold (unmeasured)
import jax
import jax.numpy as jnp
from jax import lax
from jax.experimental import pallas as pl
from jax.experimental.pallas import tpu as pltpu


def kernel(
    x,
):
    def body(*refs):
        pass

    out_shape = jax.ShapeDtypeStruct(..., jnp.float32)
    return pl.pallas_call(body, out_shape=out_shape)(...)


# baseline (device time: 230446 ns/iter reference)
import jax
import jax.numpy as jnp
from jax import lax
from jax.experimental import pallas as pl
from jax.experimental.pallas import tpu as pltpu

N_DEV = 16
M, N = 2048, 1024
CHUNK = M // N_DEV


def _ring_pos(x, y, z):
    q = 2 * x + (x ^ y)
    zz = jnp.where(q % 2 == 0, z, 3 - z)
    return 4 * q + zz


def _ring_coords(p):
    p = p % N_DEV
    q = p // 4
    zz = p % 4
    x = q // 2
    y = x ^ (q % 2)
    z = jnp.where(q % 2 == 0, zz, 3 - zz)
    return (x, y, z)


def kernel(x):
    x = x.reshape(M, N)

    def body(x_ref, out_ref, recv_buf, rs_send, rs_recv, ag_send, ag_recv):
        my_x = lax.axis_index("x")
        my_y = lax.axis_index("y")
        my_z = lax.axis_index("z")
        p = _ring_pos(my_x, my_y, my_z)
        right = _ring_coords(p + 1)
        left = _ring_coords(p - 1)

        barrier_sem = pltpu.get_barrier_semaphore()
        for nbr in (left, right):
            pl.semaphore_signal(
                barrier_sem, inc=1,
                device_id=nbr, device_id_type=pl.DeviceIdType.MESH,
            )
        pl.semaphore_wait(barrier_sem, 2)

        out_ref[:, :] = x_ref[:, :]

        for h in range(N_DEV - 1):
            soff = ((p - h) % N_DEV) * CHUNK
            rdma = pltpu.make_async_remote_copy(
                src_ref=out_ref.at[pl.ds(soff, CHUNK), :],
                dst_ref=recv_buf.at[h],
                send_sem=rs_send.at[h],
                recv_sem=rs_recv.at[h],
                device_id=right,
                device_id_type=pl.DeviceIdType.MESH,
            )
            rdma.start()
            rdma.wait()
            roff = ((p - h - 1) % N_DEV) * CHUNK
            out_ref[pl.ds(roff, CHUNK), :] = (
                out_ref[pl.ds(roff, CHUNK), :] + recv_buf[h]
            )

        for h in range(N_DEV - 1):
            soff = ((p + 1 - h) % N_DEV) * CHUNK
            rdma = pltpu.make_async_remote_copy(
                src_ref=out_ref.at[pl.ds(soff, CHUNK), :],
                dst_ref=out_ref.at[pl.ds(soff, CHUNK), :],
                send_sem=ag_send.at[h],
                recv_sem=ag_recv.at[h],
                device_id=right,
                device_id_type=pl.DeviceIdType.MESH,
            )
            rdma.start()
            rdma.wait()

    out_shape = jax.ShapeDtypeStruct((M, N), jnp.float32)
    return pl.pallas_call(
        body,
        out_shape=out_shape,
        in_specs=[pl.BlockSpec(memory_space=pltpu.VMEM)],
        out_specs=pl.BlockSpec(memory_space=pltpu.VMEM),
        scratch_shapes=[
            pltpu.VMEM((N_DEV - 1, CHUNK, N), jnp.float32),
            pltpu.SemaphoreType.DMA((N_DEV - 1,)),
            pltpu.SemaphoreType.DMA((N_DEV - 1,)),
            pltpu.SemaphoreType.DMA((N_DEV - 1,)),
            pltpu.SemaphoreType.DMA((N_DEV - 1,)),
        ],
        compiler_params=pltpu.CompilerParams(collective_id=0),
    )(x)


# device time: 148242 ns/iter; 1.5545x vs baseline; 1.5545x over previous
import jax
import jax.numpy as jnp
from jax import lax
from jax.experimental import pallas as pl
from jax.experimental.pallas import tpu as pltpu

N_DEV = 16
M, N = 2048, 1024
HALF = M // 2
CHUNK = HALF // N_DEV


def _ring_pos(x, y, z):
    q = 2 * x + (x ^ y)
    zz = jnp.where(q % 2 == 0, z, 3 - z)
    return 4 * q + zz


def _ring_coords(p):
    p = p % N_DEV
    q = p // 4
    zz = p % 4
    x = q // 2
    y = x ^ (q % 2)
    z = jnp.where(q % 2 == 0, zz, 3 - zz)
    return (x, y, z)


def kernel(x):
    x = x.reshape(M, N)

    def body(x_ref, out_ref, recv_r, recv_l,
             rs_send_r, rs_recv_r, ag_send_r, ag_recv_r,
             rs_send_l, rs_recv_l, ag_send_l, ag_recv_l):
        my_x = lax.axis_index("x")
        my_y = lax.axis_index("y")
        my_z = lax.axis_index("z")
        p = _ring_pos(my_x, my_y, my_z)
        right = _ring_coords(p + 1)
        left = _ring_coords(p - 1)
        pr = (N_DEV - p) % N_DEV

        barrier_sem = pltpu.get_barrier_semaphore()
        for nbr in (left, right):
            pl.semaphore_signal(
                barrier_sem, inc=1,
                device_id=nbr, device_id_type=pl.DeviceIdType.MESH,
            )
        pl.semaphore_wait(barrier_sem, 2)

        out_ref[:, :] = x_ref[:, :]

        def row_r(c):
            return (c % N_DEV) * CHUNK

        def row_l(c):
            return HALF + (c % N_DEV) * CHUNK

        for h in range(N_DEV - 1):
            rd_r = pltpu.make_async_remote_copy(
                src_ref=out_ref.at[pl.ds(row_r(p - h), CHUNK), :],
                dst_ref=recv_r.at[h],
                send_sem=rs_send_r.at[h],
                recv_sem=rs_recv_r.at[h],
                device_id=right,
                device_id_type=pl.DeviceIdType.MESH,
            )
            rd_l = pltpu.make_async_remote_copy(
                src_ref=out_ref.at[pl.ds(row_l(pr - h), CHUNK), :],
                dst_ref=recv_l.at[h],
                send_sem=rs_send_l.at[h],
                recv_sem=rs_recv_l.at[h],
                device_id=left,
                device_id_type=pl.DeviceIdType.MESH,
            )
            rd_r.start()
            rd_l.start()
            rd_r.wait()
            rd_l.wait()
            ro_r = row_r(p - h - 1)
            ro_l = row_l(pr - h - 1)
            out_ref[pl.ds(ro_r, CHUNK), :] = (
                out_ref[pl.ds(ro_r, CHUNK), :] + recv_r[h]
            )
            out_ref[pl.ds(ro_l, CHUNK), :] = (
                out_ref[pl.ds(ro_l, CHUNK), :] + recv_l[h]
            )

        for h in range(N_DEV - 1):
            so_r = row_r(p + 1 - h)
            so_l = row_l(pr + 1 - h)
            rd_r = pltpu.make_async_remote_copy(
                src_ref=out_ref.at[pl.ds(so_r, CHUNK), :],
                dst_ref=out_ref.at[pl.ds(so_r, CHUNK), :],
                send_sem=ag_send_r.at[h],
                recv_sem=ag_recv_r.at[h],
                device_id=right,
                device_id_type=pl.DeviceIdType.MESH,
            )
            rd_l = pltpu.make_async_remote_copy(
                src_ref=out_ref.at[pl.ds(so_l, CHUNK), :],
                dst_ref=out_ref.at[pl.ds(so_l, CHUNK), :],
                send_sem=ag_send_l.at[h],
                recv_sem=ag_recv_l.at[h],
                device_id=left,
                device_id_type=pl.DeviceIdType.MESH,
            )
            rd_r.start()
            rd_l.start()
            rd_r.wait()
            rd_l.wait()

    out_shape = jax.ShapeDtypeStruct((M, N), jnp.float32)
    return pl.pallas_call(
        body,
        out_shape=out_shape,
        in_specs=[pl.BlockSpec(memory_space=pltpu.VMEM)],
        out_specs=pl.BlockSpec(memory_space=pltpu.VMEM),
        scratch_shapes=[
            pltpu.VMEM((N_DEV - 1, CHUNK, N), jnp.float32),
            pltpu.VMEM((N_DEV - 1, CHUNK, N), jnp.float32),
            pltpu.SemaphoreType.DMA((N_DEV - 1,)),
            pltpu.SemaphoreType.DMA((N_DEV - 1,)),
            pltpu.SemaphoreType.DMA((N_DEV - 1,)),
            pltpu.SemaphoreType.DMA((N_DEV - 1,)),
            pltpu.SemaphoreType.DMA((N_DEV - 1,)),
            pltpu.SemaphoreType.DMA((N_DEV - 1,)),
            pltpu.SemaphoreType.DMA((N_DEV - 1,)),
            pltpu.SemaphoreType.DMA((N_DEV - 1,)),
        ],
        compiler_params=pltpu.CompilerParams(collective_id=0),
    )(x)


# device time: 103296 ns/iter; 2.2309x vs baseline; 1.4351x over previous
import jax
import jax.numpy as jnp
from jax import lax
from jax.experimental import pallas as pl
from jax.experimental.pallas import tpu as pltpu

N_DEV = 16
M, N = 2048, 1024
HALF = M // 2
CHUNK = HALF // N_DEV
S = 4
SUB = CHUNK // S
HOPS = N_DEV - 1


def _ring_pos(x, y, z):
    q = 2 * x + (x ^ y)
    zz = jnp.where(q % 2 == 0, z, 3 - z)
    return 4 * q + zz


def _ring_coords(p):
    p = p % N_DEV
    q = p // 4
    zz = p % 4
    x = q // 2
    y = x ^ (q % 2)
    z = jnp.where(q % 2 == 0, zz, 3 - zz)
    return (x, y, z)


def kernel(x):
    x = x.reshape(M, N)

    def body(x_ref, out_ref, recv_r, recv_l,
             rs_send_r, rs_recv_r, ag_send_r, ag_recv_r,
             rs_send_l, rs_recv_l, ag_send_l, ag_recv_l):
        my_x = lax.axis_index("x")
        my_y = lax.axis_index("y")
        my_z = lax.axis_index("z")
        p = _ring_pos(my_x, my_y, my_z)
        right = _ring_coords(p + 1)
        left = _ring_coords(p - 1)
        pr = (N_DEV - p) % N_DEV

        dirs = (
            (p, right, recv_r, rs_send_r, rs_recv_r, ag_send_r, ag_recv_r, 0),
            (pr, left, recv_l, rs_send_l, rs_recv_l, ag_send_l, ag_recv_l, HALF),
        )

        def chunk_row(d, c):
            return dirs[d][7] + (c % N_DEV) * CHUNK

        barrier_sem = pltpu.get_barrier_semaphore()
        for nbr in (left, right):
            pl.semaphore_signal(
                barrier_sem, inc=1,
                device_id=nbr, device_id_type=pl.DeviceIdType.MESH,
            )
        pl.semaphore_wait(barrier_sem, 2)

        out_ref[:, :] = x_ref[:, :]

        all_descs = []

        def rs_desc(h, s, d):
            pos, nbr, rbuf, ssem, rsem = dirs[d][:5]
            soff = chunk_row(d, pos - h) + s * SUB
            r = pltpu.make_async_remote_copy(
                src_ref=out_ref.at[pl.ds(soff, SUB), :],
                dst_ref=rbuf.at[h, pl.ds(s * SUB, SUB), :],
                send_sem=ssem.at[h, s],
                recv_sem=rsem.at[h, s],
                device_id=nbr,
                device_id_type=pl.DeviceIdType.MESH,
            )
            all_descs.append(r)
            return r

        def ag_desc(h, s, d):
            pos, nbr = dirs[d][:2]
            ssem, rsem = dirs[d][5:7]
            soff = chunk_row(d, pos + 1 - h) + s * SUB
            r = pltpu.make_async_remote_copy(
                src_ref=out_ref.at[pl.ds(soff, SUB), :],
                dst_ref=out_ref.at[pl.ds(soff, SUB), :],
                send_sem=ssem.at[h, s],
                recv_sem=rsem.at[h, s],
                device_id=nbr,
                device_id_type=pl.DeviceIdType.MESH,
            )
            all_descs.append(r)
            return r

        live = {}
        for s in range(S):
            for d in (0, 1):
                rd = rs_desc(0, s, d)
                rd.start()
                live[(s, d)] = rd
        for h in range(HOPS):
            for s in range(S):
                for d in (0, 1):
                    live[(s, d)].wait_recv()
                    pos, _, rbuf = dirs[d][:3]
                    ro = chunk_row(d, pos - h - 1) + s * SUB
                    out_ref[pl.ds(ro, SUB), :] = (
                        out_ref[pl.ds(ro, SUB), :]
                        + rbuf[h, pl.ds(s * SUB, SUB), :]
                    )
                    if h + 1 < HOPS:
                        rd = rs_desc(h + 1, s, d)
                        rd.start()
                        live[(s, d)] = rd

        for s in range(S):
            for d in (0, 1):
                rd = ag_desc(0, s, d)
                rd.start()
                live[(s, d)] = rd
        for h in range(HOPS):
            for s in range(S):
                for d in (0, 1):
                    live[(s, d)].wait_recv()
                    if h + 1 < HOPS:
                        rd = ag_desc(h + 1, s, d)
                        rd.start()
                        live[(s, d)] = rd

        for r in all_descs:
            r.wait_send()

    out_shape = jax.ShapeDtypeStruct((M, N), jnp.float32)
    return pl.pallas_call(
        body,
        out_shape=out_shape,
        in_specs=[pl.BlockSpec(memory_space=pltpu.VMEM)],
        out_specs=pl.BlockSpec(memory_space=pltpu.VMEM),
        scratch_shapes=[
            pltpu.VMEM((HOPS, CHUNK, N), jnp.float32),
            pltpu.VMEM((HOPS, CHUNK, N), jnp.float32),
            pltpu.SemaphoreType.DMA((HOPS, S)),
            pltpu.SemaphoreType.DMA((HOPS, S)),
            pltpu.SemaphoreType.DMA((HOPS, S)),
            pltpu.SemaphoreType.DMA((HOPS, S)),
            pltpu.SemaphoreType.DMA((HOPS, S)),
            pltpu.SemaphoreType.DMA((HOPS, S)),
            pltpu.SemaphoreType.DMA((HOPS, S)),
            pltpu.SemaphoreType.DMA((HOPS, S)),
        ],
        compiler_params=pltpu.CompilerParams(collective_id=0),
    )(x)


# device time: 100672 ns/iter; 2.2891x vs baseline; 1.0261x over previous
import jax
import jax.numpy as jnp
from jax import lax
from jax.experimental import pallas as pl
from jax.experimental.pallas import tpu as pltpu

N_DEV = 16
M, N = 2048, 1024
HALF = M // 2
CHUNK = HALF // N_DEV
S = 4
SUB = CHUNK // S
HOPS = N_DEV - 1


def _ring_pos(x, y, z):
    q = 2 * x + (x ^ y)
    zz = jnp.where(q % 2 == 0, z, 3 - z)
    return 4 * q + zz


def _ring_coords(p):
    p = p % N_DEV
    q = p // 4
    zz = p % 4
    x = q // 2
    y = x ^ (q % 2)
    z = jnp.where(q % 2 == 0, zz, 3 - zz)
    return (x, y, z)


def kernel(x):
    x = x.reshape(M, N)

    def body(x_ref, out_ref, recv_r, recv_l,
             rs_send_r, rs_recv_r, ag_send_r, ag_recv_r,
             rs_send_l, rs_recv_l, ag_send_l, ag_recv_l):
        my_x = lax.axis_index("x")
        my_y = lax.axis_index("y")
        my_z = lax.axis_index("z")
        p = _ring_pos(my_x, my_y, my_z)
        right = _ring_coords(p + 1)
        left = _ring_coords(p - 1)
        pr = (N_DEV - p) % N_DEV

        dirs = (
            (p, right, recv_r, rs_send_r, rs_recv_r, ag_send_r, ag_recv_r, 0),
            (pr, left, recv_l, rs_send_l, rs_recv_l, ag_send_l, ag_recv_l, HALF),
        )

        def chunk_row(d, c):
            return dirs[d][7] + (c % N_DEV) * CHUNK

        barrier_sem = pltpu.get_barrier_semaphore()
        for nbr in (left, right):
            pl.semaphore_signal(
                barrier_sem, inc=1,
                device_id=nbr, device_id_type=pl.DeviceIdType.MESH,
            )
        pl.semaphore_wait(barrier_sem, 2)

        all_descs = []

        def rs_desc(h, s, d):
            pos, nbr, rbuf, ssem, rsem = dirs[d][:5]
            soff = chunk_row(d, pos - h) + s * SUB
            src = x_ref if h == 0 else out_ref
            r = pltpu.make_async_remote_copy(
                src_ref=src.at[pl.ds(soff, SUB), :],
                dst_ref=rbuf.at[h, pl.ds(s * SUB, SUB), :],
                send_sem=ssem.at[h, s],
                recv_sem=rsem.at[h, s],
                device_id=nbr,
                device_id_type=pl.DeviceIdType.MESH,
            )
            all_descs.append(r)
            return r

        def ag_desc(h, s, d):
            pos, nbr = dirs[d][:2]
            ssem, rsem = dirs[d][5:7]
            soff = chunk_row(d, pos + 1 - h) + s * SUB
            r = pltpu.make_async_remote_copy(
                src_ref=out_ref.at[pl.ds(soff, SUB), :],
                dst_ref=out_ref.at[pl.ds(soff, SUB), :],
                send_sem=ssem.at[h, s],
                recv_sem=rsem.at[h, s],
                device_id=nbr,
                device_id_type=pl.DeviceIdType.MESH,
            )
            all_descs.append(r)
            return r

        live = {}
        for s in range(S):
            for d in (0, 1):
                rd = rs_desc(0, s, d)
                rd.start()
                live[(s, d)] = rd
        for h in range(HOPS):
            for s in range(S):
                for d in (0, 1):
                    live[(s, d)].wait_recv()
                    pos, _, rbuf = dirs[d][:3]
                    ro = chunk_row(d, pos - h - 1) + s * SUB
                    out_ref[pl.ds(ro, SUB), :] = (
                        x_ref[pl.ds(ro, SUB), :]
                        + rbuf[h, pl.ds(s * SUB, SUB), :]
                    )
                    if h + 1 < HOPS:
                        rd = rs_desc(h + 1, s, d)
                        rd.start()
                        live[(s, d)] = rd
                    else:
                        rd = ag_desc(0, s, d)
                        rd.start()
                        live[(s, d)] = rd

        for h in range(HOPS):
            for s in range(S):
                for d in (0, 1):
                    live[(s, d)].wait_recv()
                    if h + 1 < HOPS:
                        rd = ag_desc(h + 1, s, d)
                        rd.start()
                        live[(s, d)] = rd

        for r in all_descs:
            r.wait_send()

    out_shape = jax.ShapeDtypeStruct((M, N), jnp.float32)
    return pl.pallas_call(
        body,
        out_shape=out_shape,
        in_specs=[pl.BlockSpec(memory_space=pltpu.VMEM)],
        out_specs=pl.BlockSpec(memory_space=pltpu.VMEM),
        scratch_shapes=[
            pltpu.VMEM((HOPS, CHUNK, N), jnp.float32),
            pltpu.VMEM((HOPS, CHUNK, N), jnp.float32),
            pltpu.SemaphoreType.DMA((HOPS, S)),
            pltpu.SemaphoreType.DMA((HOPS, S)),
            pltpu.SemaphoreType.DMA((HOPS, S)),
            pltpu.SemaphoreType.DMA((HOPS, S)),
            pltpu.SemaphoreType.DMA((HOPS, S)),
            pltpu.SemaphoreType.DMA((HOPS, S)),
            pltpu.SemaphoreType.DMA((HOPS, S)),
            pltpu.SemaphoreType.DMA((HOPS, S)),
        ],
        compiler_params=pltpu.CompilerParams(collective_id=0),
    )(x)
